# Initial kernel scaffold; baseline (speedup 1.0000x reference)
import jax
import jax.numpy as jnp
from jax.experimental import pallas as pl


def kernel(x, edge_index, graph_has_graphlet, W1a, b1a, W2a, b2a, W1b, b1b, W2b, b2b):
    raise NotImplementedError("write your pallas kernel here")


# speedup vs baseline: 5.8155x; 5.8155x over previous
import functools

import jax
import jax.numpy as jnp
from jax import lax
from jax.experimental import pallas as pl
from jax.experimental.pallas import tpu as pltpu
from jax.experimental.pallas import tpu_sc as plsc

N = 10000
E = 320000
D = 128
G = 64
GSZ = 5

NC = 2
NS = 16
NW = NC * NS
CHUNK = 128
NCHUNKS = E // CHUNK
RB = 200
NB = N // RB


def _sc_agg_body(table_hbm, src_hbm, dst_hbm, out_hbm,
                 src_v, dst_v, rows_v, zrow_v, acc_sh, sem):
    c = lax.axis_index("c")
    s = lax.axis_index("s")
    w = s * NC + c

    zero16 = jnp.zeros((16,), jnp.float32)

    def _zero_body(i, carry):
        for j in range(D // 16):
            zrow_v[i, pl.ds(j * 16, 16)] = zero16
        return carry

    lax.fori_loop(0, RB, _zero_body, 0)

    nb_loops = (NB + NS - 1) // NS

    def _zinit(k, carry):
        bid = s + NS * k

        @pl.when(bid < NB)
        def _():
            off = pl.multiple_of(bid * RB, 8)
            pltpu.sync_copy(zrow_v, acc_sh.at[pl.ds(off, RB)])

        return carry

    lax.fori_loop(0, nb_loops, _zinit, 0)
    plsc.subcore_barrier()

    nloops = (NCHUNKS + NW - 1) // NW

    def _edge_body(i, carry):
        cid = w + i * NW

        @pl.when(cid < NCHUNKS)
        def _():
            base = cid * CHUNK
            pltpu.sync_copy(src_hbm.at[pl.ds(base, CHUNK)], src_v)
            pltpu.sync_copy(dst_hbm.at[pl.ds(base, CHUNK)], dst_v)
            pltpu.async_copy(table_hbm.at[src_v], rows_v, sem).wait()
            pltpu.sync_copy(rows_v, acc_sh.at[dst_v], add=True)

        return carry

    lax.fori_loop(0, nloops, _edge_body, 0)
    plsc.subcore_barrier()

    def _dump(k, carry):
        bid = s + NS * k

        @pl.when(bid < NB)
        def _():
            off = pl.multiple_of(bid * RB, 8)
            pltpu.sync_copy(acc_sh.at[pl.ds(off, RB)],
                            out_hbm.at[c, pl.ds(off, RB)])

        return carry

    lax.fori_loop(0, nb_loops, _dump, 0)


def _sc_pass(table, src, dst):
    mesh = plsc.VectorSubcoreMesh(core_axis_name="c", subcore_axis_name="s")
    kern = pl.kernel(
        _sc_agg_body,
        mesh=mesh,
        out_type=jax.ShapeDtypeStruct((NC, N, D), jnp.float32),
        scratch_types=[
            pltpu.VMEM((CHUNK,), jnp.int32),
            pltpu.VMEM((CHUNK,), jnp.int32),
            pltpu.VMEM((CHUNK, D), jnp.float32),
            pltpu.VMEM((RB, D), jnp.float32),
            pltpu.VMEM_SHARED((N, D), jnp.float32),
            pltpu.SemaphoreType.DMA,
        ],
    )
    return kern(table, src, dst)


ROWS_BLK = 1000
GRID = N // ROWS_BLK


def _mlp1_body(x_ref, pa_ref, w1_ref, b1_ref, w2_ref, b2_ref, out_ref):
    h = x_ref[...] + pa_ref[0] + pa_ref[1]
    t = jnp.maximum(
        jnp.dot(h, w1_ref[...], preferred_element_type=jnp.float32)
        + b1_ref[...], 0.0)
    o = (jnp.dot(t, w2_ref[...], preferred_element_type=jnp.float32)
         + b2_ref[...])
    out_ref[...] = jnp.maximum(o, 0.0)


def _mlp1(x, pa, w1, b1, w2, b2):
    return pl.pallas_call(
        _mlp1_body,
        grid=(GRID,),
        in_specs=[
            pl.BlockSpec((ROWS_BLK, D), lambda i: (i, 0)),
            pl.BlockSpec((NC, ROWS_BLK, D), lambda i: (0, i, 0)),
            pl.BlockSpec((D, D), lambda i: (0, 0)),
            pl.BlockSpec((1, D), lambda i: (0, 0)),
            pl.BlockSpec((D, D), lambda i: (0, 0)),
            pl.BlockSpec((1, D), lambda i: (0, 0)),
        ],
        out_specs=pl.BlockSpec((ROWS_BLK, D), lambda i: (i, 0)),
        out_shape=jax.ShapeDtypeStruct((N, D), jnp.float32),
    )(x, pa, w1, b1, w2, b2)


def _mlp2_body(h_ref, pa_ref, w1_ref, b1_ref, w2_ref, b2_ref,
               e_ref, g_ref, out_ref):
    i = pl.program_id(0)
    hin = h_ref[...] + pa_ref[0] + pa_ref[1]
    t = jnp.maximum(
        jnp.dot(hin, w1_ref[...], preferred_element_type=jnp.float32)
        + b1_ref[...], 0.0)
    h2 = (jnp.dot(t, w2_ref[...], preferred_element_type=jnp.float32)
          + b2_ref[...])
    contrib = lax.dot_general(e_ref[...], h2, (((0,), (0,)), ((), ())),
                              preferred_element_type=jnp.float32)

    @pl.when(i == 0)
    def _():
        out_ref[...] = jnp.zeros_like(out_ref)

    out_ref[...] += contrib

    @pl.when(i == pl.num_programs(0) - 1)
    def _():
        den = jnp.sum(g_ref[...], axis=1, keepdims=True) + 1e-4
        out_ref[...] = out_ref[...] / den


def _mlp2(h, pa, w1, b1, w2, b2, e_rep, ghg):
    return pl.pallas_call(
        _mlp2_body,
        grid=(GRID,),
        in_specs=[
            pl.BlockSpec((ROWS_BLK, D), lambda i: (i, 0)),
            pl.BlockSpec((NC, ROWS_BLK, D), lambda i: (0, i, 0)),
            pl.BlockSpec((D, D), lambda i: (0, 0)),
            pl.BlockSpec((1, D), lambda i: (0, 0)),
            pl.BlockSpec((D, D), lambda i: (0, 0)),
            pl.BlockSpec((1, D), lambda i: (0, 0)),
            pl.BlockSpec((ROWS_BLK, G), lambda i: (i, 0)),
            pl.BlockSpec((G, N // GSZ), lambda i: (0, 0)),
        ],
        out_specs=pl.BlockSpec((G, D), lambda i: (0, 0)),
        out_shape=jax.ShapeDtypeStruct((G, D), jnp.float32),
    )(h, pa, w1, b1, w2, b2, e_rep, ghg)


def kernel(x, edge_index, graph_has_graphlet,
           W1a, b1a, W2a, b2a, W1b, b1b, W2b, b2b):
    src = edge_index[0]
    dst = edge_index[1]
    b1a2 = b1a.reshape(1, D)
    b2a2 = b2a.reshape(1, D)
    b1b2 = b1b.reshape(1, D)
    b2b2 = b2b.reshape(1, D)
    ghg_rep_t = jnp.repeat(graph_has_graphlet.T, GSZ, axis=0)

    pa1 = _sc_pass(x, src, dst)
    h1r = _mlp1(x, pa1, W1a, b1a2, W2a, b2a2)
    pa2 = _sc_pass(h1r, src, dst)
    out = _mlp2(h1r, pa2, W1b, b1b2, W2b, b2b2, ghg_rep_t, graph_has_graphlet)
    return out
